# baseline (speedup 1.0000x reference)
import functools

import jax
import jax.numpy as jnp
from jax import lax
from jax.experimental import pallas as pl
from jax.experimental.pallas import tpu as pltpu
from jax.experimental.pallas import tpu_sc as plsc

ATOM_FEA = 128
NBR_FEA = 16
N_NODES = 10000
N_EDGES = 320000
OUT_DIM = 2 * ATOM_FEA
EPS = 1e-5

NC, NS = 2, 16
NW = NC * NS
E_PER_W = N_EDGES // NW

GB = 40
N_GCHUNK = E_PER_W // GB
SB = 80
N_SCHUNK = E_PER_W // SB

NODE_BLK = 1000
EB = 2000
N_EBLK = N_EDGES // EB

def _mesh():
    return plsc.VectorSubcoreMesh(core_axis_name="c", subcore_axis_name="s",
                                  num_cores=NC, num_subcores=NS)


def _prep_body(a_ref, w_ref, ps_ref, pd_ref):
    a = a_ref[...]
    ws = w_ref[:, :ATOM_FEA]
    wd = w_ref[:, ATOM_FEA:2 * ATOM_FEA]
    dn = (((1,), (1,)), ((), ()))
    ps_ref[...] = lax.dot_general(a, ws, dn, preferred_element_type=jnp.float32)
    pd_ref[...] = lax.dot_general(a, wd, dn, preferred_element_type=jnp.float32)


def _prep(atom_in_fea, w):
    return pl.pallas_call(
        _prep_body,
        grid=(N_NODES // NODE_BLK,),
        in_specs=[
            pl.BlockSpec((NODE_BLK, ATOM_FEA), lambda i: (i, 0)),
            pl.BlockSpec((OUT_DIM, 2 * ATOM_FEA + NBR_FEA), lambda i: (0, 0)),
        ],
        out_specs=[
            pl.BlockSpec((NODE_BLK, OUT_DIM), lambda i: (i, 0)),
            pl.BlockSpec((NODE_BLK, OUT_DIM), lambda i: (i, 0)),
        ],
        out_shape=[jax.ShapeDtypeStruct((N_NODES, OUT_DIM), jnp.float32)] * 2,
    )(atom_in_fea, w)


def _gather_body(ps_hbm, pd_hbm, src_hbm, dst_hbm, out_hbm,
                 si_v, di_v, rs, rd, ob, sem_s, sem_d, sem_o):
    wid = lax.axis_index("s") * NC + lax.axis_index("c")
    base0 = wid * E_PER_W
    pltpu.sync_copy(src_hbm.at[pl.ds(base0, E_PER_W)], si_v)
    pltpu.sync_copy(dst_hbm.at[pl.ds(base0, E_PER_W)], di_v)

    def issue(j, p):
        off = j * GB
        pltpu.async_copy(ps_hbm.at[si_v.at[pl.ds(off, GB)]], rs.at[p], sem_s[p])
        pltpu.async_copy(pd_hbm.at[di_v.at[pl.ds(off, GB)]], rd.at[p], sem_d[p])

    issue(0, 0)
    issue(1, 1)

    def half(k, p):
        j = 2 * k + p
        pltpu.make_async_copy(ps_hbm.at[si_v.at[pl.ds(0, GB)]],
                              rs.at[p], sem_s[p]).wait()
        pltpu.make_async_copy(pd_hbm.at[di_v.at[pl.ds(0, GB)]],
                              rd.at[p], sem_d[p]).wait()

        def addrow(b, c2):
            for c in range(OUT_DIM // 16):
                sl = pl.ds(c * 16, 16)
                ob[p, b, sl] = rs[p, b, sl] + rd[p, b, sl]
            return c2
        lax.fori_loop(0, GB, addrow, 0, unroll=4)

        @pl.when(j + 2 < N_GCHUNK)
        def _():
            issue(j + 2, p)

        @pl.when(k > 0)
        def _():
            pltpu.make_async_copy(ob.at[p], out_hbm.at[pl.ds(0, GB)],
                                  sem_o[p]).wait()
        pltpu.async_copy(ob.at[p], out_hbm.at[pl.ds(base0 + j * GB, GB)],
                         sem_o[p])

    def pair(k, carry):
        half(k, 0)
        half(k, 1)
        return carry

    lax.fori_loop(0, N_GCHUNK // 2, pair, 0)
    pltpu.make_async_copy(ob.at[0], out_hbm.at[pl.ds(0, GB)], sem_o[0]).wait()
    pltpu.make_async_copy(ob.at[1], out_hbm.at[pl.ds(0, GB)], sem_o[1]).wait()


def _sc_gather(ps, pd, src_i, dst_i):
    f = pl.kernel(
        _gather_body,
        out_type=jax.ShapeDtypeStruct((N_EDGES, OUT_DIM), jnp.float32),
        mesh=_mesh(),
        scratch_types=[
            pltpu.VMEM((E_PER_W,), jnp.int32),
            pltpu.VMEM((E_PER_W,), jnp.int32),
            pltpu.VMEM((2, GB, OUT_DIM), jnp.float32),
            pltpu.VMEM((2, GB, OUT_DIM), jnp.float32),
            pltpu.VMEM((2, GB, OUT_DIM), jnp.float32),
            [pltpu.SemaphoreType.DMA, pltpu.SemaphoreType.DMA],
            [pltpu.SemaphoreType.DMA, pltpu.SemaphoreType.DMA],
            [pltpu.SemaphoreType.DMA, pltpu.SemaphoreType.DMA],
        ],
    )
    return f(ps, pd, src_i, dst_i)


def _stats_body(xp_ref, nbr_ref, wn_ref, b_ref, s_ref):
    i = pl.program_id(0)
    dn = (((1,), (1,)), ((), ()))
    nproj = lax.dot_general(nbr_ref[...], wn_ref[...], dn,
                            preferred_element_type=jnp.float32)
    x = xp_ref[...] + nproj + b_ref[...]
    blk = jnp.concatenate(
        [jnp.sum(x, axis=0, keepdims=True),
         jnp.sum(x * x, axis=0, keepdims=True)], axis=0)

    @pl.when(i == 0)
    def _():
        s_ref[...] = blk

    @pl.when(i > 0)
    def _():
        s_ref[...] = s_ref[...] + blk


def _stats(xpre, nbr, wn, b2d):
    return pl.pallas_call(
        _stats_body,
        grid=(N_EBLK,),
        in_specs=[
            pl.BlockSpec((EB, OUT_DIM), lambda i: (i, 0)),
            pl.BlockSpec((EB, NBR_FEA), lambda i: (i, 0)),
            pl.BlockSpec((OUT_DIM, NBR_FEA), lambda i: (0, 0)),
            pl.BlockSpec((1, OUT_DIM), lambda i: (0, 0)),
        ],
        out_specs=pl.BlockSpec((2, OUT_DIM), lambda i: (0, 0)),
        out_shape=jax.ShapeDtypeStruct((2, OUT_DIM), jnp.float32),
    )(xpre, nbr, wn, b2d)


def _normact_body(xp_ref, nbr_ref, wn_ref, a1_ref, c1_ref, msg_ref):
    dn = (((1,), (1,)), ((), ()))
    nproj = lax.dot_general(nbr_ref[...], wn_ref[...], dn,
                            preferred_element_type=jnp.float32)
    y = (xp_ref[...] + nproj) * a1_ref[...] + c1_ref[...]
    f = y[:, :ATOM_FEA]
    c = y[:, ATOM_FEA:]
    tf = jnp.exp(-jnp.abs(f))
    sig = jnp.where(f >= 0, 1.0 / (1.0 + tf), tf / (1.0 + tf))
    sp = jnp.maximum(c, 0.0) + jnp.log(1.0 + jnp.exp(-jnp.abs(c)))
    msg_ref[...] = sig * sp


def _normact(xpre, nbr, wn, a1, c1):
    return pl.pallas_call(
        _normact_body,
        grid=(N_EBLK,),
        in_specs=[
            pl.BlockSpec((EB, OUT_DIM), lambda i: (i, 0)),
            pl.BlockSpec((EB, NBR_FEA), lambda i: (i, 0)),
            pl.BlockSpec((OUT_DIM, NBR_FEA), lambda i: (0, 0)),
            pl.BlockSpec((1, OUT_DIM), lambda i: (0, 0)),
            pl.BlockSpec((1, OUT_DIM), lambda i: (0, 0)),
        ],
        out_specs=pl.BlockSpec((EB, ATOM_FEA), lambda i: (i, 0)),
        out_shape=jax.ShapeDtypeStruct((N_EDGES, ATOM_FEA), jnp.float32),
    )(xpre, nbr, wn, a1, c1)


def _scatter_body(msg_hbm, dst_hbm, out_hbm, di2_v, rows_v, zbuf_v, acc_sp,
                  sem_m, sem_i):
    cid = lax.axis_index("c")
    tid = lax.axis_index("s")
    wid = tid * NC + cid
    base0 = wid * E_PER_W

    def zrow(r, c2):
        for c in range(ATOM_FEA // 16):
            zbuf_v[r, pl.ds(c * 16, 16)] = jnp.zeros((16,), jnp.float32)
        return c2
    lax.fori_loop(0, 200, zrow, 0)

    @pl.when(tid < 10)
    def _():
        def zc(j, c2):
            pltpu.sync_copy(zbuf_v, acc_sp.at[pl.ds(tid * 1000 + j * 200, 200)])
            return c2
        lax.fori_loop(0, 5, zc, 0)

    plsc.subcore_barrier()

    def issue(j, p):
        off = base0 + j * SB
        pltpu.async_copy(msg_hbm.at[pl.ds(off, SB)], rows_v.at[p], sem_m[p])
        pltpu.async_copy(dst_hbm.at[pl.ds(off, SB)], di2_v.at[p], sem_i[p])

    issue(0, 0)
    issue(1, 1)

    def half(k, p):
        j = 2 * k + p
        pltpu.make_async_copy(msg_hbm.at[pl.ds(0, SB)], rows_v.at[p],
                              sem_m[p]).wait()
        pltpu.make_async_copy(dst_hbm.at[pl.ds(0, SB)], di2_v.at[p],
                              sem_i[p]).wait()
        pltpu.sync_copy(rows_v.at[p], acc_sp.at[di2_v.at[p]], add=True)

        @pl.when(j + 2 < N_SCHUNK)
        def _():
            issue(j + 2, p)

    def pair(k, carry):
        half(k, 0)
        half(k, 1)
        return carry
    lax.fori_loop(0, N_SCHUNK // 2, pair, 0)
    half(N_SCHUNK // 2, 0)

    plsc.subcore_barrier()

    @pl.when(tid < 10)
    def _():
        def dc(j, c2):
            row = tid * 1000 + j * 200
            pltpu.sync_copy(acc_sp.at[pl.ds(row, 200)], zbuf_v)
            pltpu.sync_copy(zbuf_v, out_hbm.at[cid].at[pl.ds(row, 200)])
            return c2
        lax.fori_loop(0, 5, dc, 0)


def _sc_scatter(msg, dst_i):
    f = pl.kernel(
        _scatter_body,
        out_type=jax.ShapeDtypeStruct((NC, N_NODES, ATOM_FEA), jnp.float32),
        mesh=_mesh(),
        scratch_types=[
            pltpu.VMEM((2, SB), jnp.int32),
            pltpu.VMEM((2, SB, ATOM_FEA), jnp.float32),
            pltpu.VMEM((200, ATOM_FEA), jnp.float32),
            pltpu.VMEM_SHARED((N_NODES, ATOM_FEA), jnp.float32),
            [pltpu.SemaphoreType.DMA, pltpu.SemaphoreType.DMA],
            [pltpu.SemaphoreType.DMA, pltpu.SemaphoreType.DMA],
        ],
    )
    return f(msg, dst_i)


def _final_body(u_ref, a_ref, g2_ref, b2_ref, out_ref):
    u = u_ref[0] + u_ref[1]
    n = jnp.float32(N_NODES)
    m = jnp.sum(u, axis=0, keepdims=True) / n
    v = jnp.sum(u * u, axis=0, keepdims=True) / n - m * m
    un = (u - m) * lax.rsqrt(v + EPS) * g2_ref[...] + b2_ref[...]
    z = a_ref[...] + un
    out_ref[...] = jnp.maximum(z, 0.0) + jnp.log(1.0 + jnp.exp(-jnp.abs(z)))


def _final(upd2, atom_in_fea, g2, b2):
    return pl.pallas_call(
        _final_body,
        grid=(1,),
        in_specs=[
            pl.BlockSpec((NC, N_NODES, ATOM_FEA), lambda i: (0, 0, 0)),
            pl.BlockSpec((N_NODES, ATOM_FEA), lambda i: (0, 0)),
            pl.BlockSpec((1, ATOM_FEA), lambda i: (0, 0)),
            pl.BlockSpec((1, ATOM_FEA), lambda i: (0, 0)),
        ],
        out_specs=pl.BlockSpec((N_NODES, ATOM_FEA), lambda i: (0, 0)),
        out_shape=jax.ShapeDtypeStruct((N_NODES, ATOM_FEA), jnp.float32),
    )(upd2, atom_in_fea, g2, b2)


def kernel(atom_in_fea, nbr_fea, edge_src, edge_dst, W, b,
           gamma1, beta1, gamma2, beta2):
    src_i = edge_src.astype(jnp.int32)
    dst_i = edge_dst.astype(jnp.int32)
    wn = W[:, 2 * ATOM_FEA:]

    ps, pd = _prep(atom_in_fea, W)
    xpre = _sc_gather(ps, pd, src_i, dst_i)
    sums = _stats(xpre, nbr_fea, wn, b.reshape(1, OUT_DIM))

    e = jnp.float32(N_EDGES)
    mean = sums[0] / e
    var = sums[1] / e - mean * mean
    a1 = gamma1 * lax.rsqrt(var + EPS)
    c1 = beta1 + (b - mean) * a1

    msg = _normact(xpre, nbr_fea, wn, a1.reshape(1, OUT_DIM),
                   c1.reshape(1, OUT_DIM))
    upd2 = _sc_scatter(msg, dst_i)
    return _final(upd2, atom_in_fea, gamma2.reshape(1, ATOM_FEA),
                  beta2.reshape(1, ATOM_FEA))


# speedup vs baseline: 1.1250x; 1.1250x over previous
import functools

import jax
import jax.numpy as jnp
from jax import lax
from jax.experimental import pallas as pl
from jax.experimental.pallas import tpu as pltpu
from jax.experimental.pallas import tpu_sc as plsc

ATOM_FEA = 128
NBR_FEA = 16
N_NODES = 10000
N_EDGES = 320000
OUT_DIM = 2 * ATOM_FEA
EPS = 1e-5

NC, NS = 2, 16
NW = NC * NS
E_HALF = N_EDGES // 2
EPW = E_HALF // NW

GB = 40
N_GCHUNK = EPW // GB
SB = 40
N_SCHUNK = EPW // SB

NODE_BLK = 1000
EB = 2000
N_EBLK_H = E_HALF // EB


def _mesh():
    return plsc.VectorSubcoreMesh(core_axis_name="c", subcore_axis_name="s",
                                  num_cores=NC, num_subcores=NS)


def _prep_body(a_ref, w_ref, ps_ref, pd_ref):
    a = a_ref[...]
    ws = w_ref[:, :ATOM_FEA]
    wd = w_ref[:, ATOM_FEA:2 * ATOM_FEA]
    dn = (((1,), (1,)), ((), ()))
    ps_ref[...] = lax.dot_general(a, ws, dn, preferred_element_type=jnp.float32)
    pd_ref[...] = lax.dot_general(a, wd, dn, preferred_element_type=jnp.float32)


def _prep(atom_in_fea, w):
    return pl.pallas_call(
        _prep_body,
        grid=(N_NODES // NODE_BLK,),
        in_specs=[
            pl.BlockSpec((NODE_BLK, ATOM_FEA), lambda i: (i, 0)),
            pl.BlockSpec((OUT_DIM, 2 * ATOM_FEA + NBR_FEA), lambda i: (0, 0)),
        ],
        out_specs=[
            pl.BlockSpec((NODE_BLK, OUT_DIM), lambda i: (i, 0)),
            pl.BlockSpec((NODE_BLK, OUT_DIM), lambda i: (i, 0)),
        ],
        out_shape=[jax.ShapeDtypeStruct((N_NODES, OUT_DIM), jnp.float32)] * 2,
    )(atom_in_fea, w)


def _gather_body(base, ps_hbm, pd_hbm, src_hbm, dst_hbm, out_hbm,
                 si_v, di_v, rs, rd, ob, sem_s, sem_d, sem_o):
    wid = lax.axis_index("s") * NC + lax.axis_index("c")
    base0 = base + wid * EPW
    pltpu.sync_copy(src_hbm.at[pl.ds(base0, EPW)], si_v)
    pltpu.sync_copy(dst_hbm.at[pl.ds(base0, EPW)], di_v)

    def issue(j, p):
        off = j * GB
        pltpu.async_copy(ps_hbm.at[si_v.at[pl.ds(off, GB)]], rs.at[p], sem_s[p])
        pltpu.async_copy(pd_hbm.at[di_v.at[pl.ds(off, GB)]], rd.at[p], sem_d[p])

    issue(0, 0)
    issue(1, 1)

    def half(k, p):
        j = 2 * k + p
        pltpu.make_async_copy(ps_hbm.at[si_v.at[pl.ds(0, GB)]],
                              rs.at[p], sem_s[p]).wait()
        pltpu.make_async_copy(pd_hbm.at[di_v.at[pl.ds(0, GB)]],
                              rd.at[p], sem_d[p]).wait()

        def addrow(b, c2):
            for c in range(OUT_DIM // 16):
                sl = pl.ds(c * 16, 16)
                ob[p, b, sl] = rs[p, b, sl] + rd[p, b, sl]
            return c2
        lax.fori_loop(0, GB, addrow, 0, unroll=2)

        @pl.when(j + 2 < N_GCHUNK)
        def _():
            issue(j + 2, p)

        @pl.when(k > 0)
        def _():
            pltpu.make_async_copy(ob.at[p], out_hbm.at[pl.ds(0, GB)],
                                  sem_o[p]).wait()
        pltpu.async_copy(ob.at[p],
                         out_hbm.at[pl.ds(wid * EPW + j * GB, GB)], sem_o[p])

    def pair(k, carry):
        half(k, 0)
        half(k, 1)
        return carry

    lax.fori_loop(0, N_GCHUNK // 2, pair, 0)
    half(N_GCHUNK // 2, 0)
    pltpu.make_async_copy(ob.at[0], out_hbm.at[pl.ds(0, GB)], sem_o[0]).wait()
    pltpu.make_async_copy(ob.at[1], out_hbm.at[pl.ds(0, GB)], sem_o[1]).wait()


def _sc_gather(base, ps, pd, src_i, dst_i):
    f = pl.kernel(
        functools.partial(_gather_body, base),
        out_type=jax.ShapeDtypeStruct((E_HALF, OUT_DIM), jnp.float32),
        mesh=_mesh(),
        scratch_types=[
            pltpu.VMEM((EPW,), jnp.int32),
            pltpu.VMEM((EPW,), jnp.int32),
            pltpu.VMEM((2, GB, OUT_DIM), jnp.float32),
            pltpu.VMEM((2, GB, OUT_DIM), jnp.float32),
            pltpu.VMEM((2, GB, OUT_DIM), jnp.float32),
            [pltpu.SemaphoreType.DMA, pltpu.SemaphoreType.DMA],
            [pltpu.SemaphoreType.DMA, pltpu.SemaphoreType.DMA],
            [pltpu.SemaphoreType.DMA, pltpu.SemaphoreType.DMA],
        ],
    )
    return f(ps, pd, src_i, dst_i)


def _stats_body(xp_ref, nbr_ref, wn_ref, b_ref, s_ref):
    i = pl.program_id(0)
    dn = (((1,), (1,)), ((), ()))
    nproj = lax.dot_general(nbr_ref[...], wn_ref[...], dn,
                            preferred_element_type=jnp.float32)
    x = xp_ref[...] + nproj + b_ref[...]
    blk = jnp.concatenate(
        [jnp.sum(x, axis=0, keepdims=True),
         jnp.sum(x * x, axis=0, keepdims=True)], axis=0)

    @pl.when(i == 0)
    def _():
        s_ref[...] = blk

    @pl.when(i > 0)
    def _():
        s_ref[...] = s_ref[...] + blk


def _stats(xpre_h, nbr, wn, b2d, off_blk):
    return pl.pallas_call(
        _stats_body,
        grid=(N_EBLK_H,),
        in_specs=[
            pl.BlockSpec((EB, OUT_DIM), lambda i: (i, 0)),
            pl.BlockSpec((EB, NBR_FEA), lambda i, o=off_blk: (i + o, 0)),
            pl.BlockSpec((OUT_DIM, NBR_FEA), lambda i: (0, 0)),
            pl.BlockSpec((1, OUT_DIM), lambda i: (0, 0)),
        ],
        out_specs=pl.BlockSpec((2, OUT_DIM), lambda i: (0, 0)),
        out_shape=jax.ShapeDtypeStruct((2, OUT_DIM), jnp.float32),
    )(xpre_h, nbr, wn, b2d)


def _normact_body(xp_ref, nbr_ref, wn_ref, a1_ref, c1_ref, msg_ref):
    dn = (((1,), (1,)), ((), ()))
    nproj = lax.dot_general(nbr_ref[...], wn_ref[...], dn,
                            preferred_element_type=jnp.float32)
    y = (xp_ref[...] + nproj) * a1_ref[...] + c1_ref[...]
    f = y[:, :ATOM_FEA]
    c = y[:, ATOM_FEA:]
    tf = jnp.exp(-jnp.abs(f))
    sig = jnp.where(f >= 0, 1.0 / (1.0 + tf), tf / (1.0 + tf))
    sp = jnp.maximum(c, 0.0) + jnp.log(1.0 + jnp.exp(-jnp.abs(c)))
    msg_ref[...] = sig * sp


def _normact(xpre_h, nbr, wn, a1, c1, off_blk):
    return pl.pallas_call(
        _normact_body,
        grid=(N_EBLK_H,),
        in_specs=[
            pl.BlockSpec((EB, OUT_DIM), lambda i: (i, 0)),
            pl.BlockSpec((EB, NBR_FEA), lambda i, o=off_blk: (i + o, 0)),
            pl.BlockSpec((OUT_DIM, NBR_FEA), lambda i: (0, 0)),
            pl.BlockSpec((1, OUT_DIM), lambda i: (0, 0)),
            pl.BlockSpec((1, OUT_DIM), lambda i: (0, 0)),
        ],
        out_specs=pl.BlockSpec((EB, ATOM_FEA), lambda i: (i, 0)),
        out_shape=jax.ShapeDtypeStruct((E_HALF, ATOM_FEA), jnp.float32),
    )(xpre_h, nbr, wn, a1, c1)


def _scatter_body(base, msg_hbm, dst_hbm, out_hbm, di2_v, rows_v, zbuf_v,
                  acc_sp, sem_m, sem_i):
    cid = lax.axis_index("c")
    tid = lax.axis_index("s")
    wid = tid * NC + cid
    mbase0 = wid * EPW
    dbase0 = base + wid * EPW

    def zrow(r, c2):
        for c in range(ATOM_FEA // 16):
            zbuf_v[r, pl.ds(c * 16, 16)] = jnp.zeros((16,), jnp.float32)
        return c2
    lax.fori_loop(0, 200, zrow, 0)

    @pl.when(tid < 10)
    def _():
        def zc(j, c2):
            pltpu.sync_copy(zbuf_v, acc_sp.at[pl.ds(tid * 1000 + j * 200, 200)])
            return c2
        lax.fori_loop(0, 5, zc, 0)

    plsc.subcore_barrier()

    def issue(j, p):
        pltpu.async_copy(msg_hbm.at[pl.ds(mbase0 + j * SB, SB)],
                         rows_v.at[p], sem_m[p])
        pltpu.async_copy(dst_hbm.at[pl.ds(dbase0 + j * SB, SB)],
                         di2_v.at[p], sem_i[p])

    issue(0, 0)
    issue(1, 1)

    def half(k, p):
        j = 2 * k + p
        pltpu.make_async_copy(msg_hbm.at[pl.ds(0, SB)], rows_v.at[p],
                              sem_m[p]).wait()
        pltpu.make_async_copy(dst_hbm.at[pl.ds(0, SB)], di2_v.at[p],
                              sem_i[p]).wait()
        pltpu.sync_copy(rows_v.at[p], acc_sp.at[di2_v.at[p]], add=True)

        @pl.when(j + 2 < N_SCHUNK)
        def _():
            issue(j + 2, p)

    def pair(k, carry):
        half(k, 0)
        half(k, 1)
        return carry
    lax.fori_loop(0, N_SCHUNK // 2, pair, 0)
    half(N_SCHUNK // 2, 0)

    plsc.subcore_barrier()

    @pl.when(tid < 10)
    def _():
        def dc(j, c2):
            row = tid * 1000 + j * 200
            pltpu.sync_copy(acc_sp.at[pl.ds(row, 200)], zbuf_v)
            pltpu.sync_copy(zbuf_v, out_hbm.at[cid].at[pl.ds(row, 200)])
            return c2
        lax.fori_loop(0, 5, dc, 0)


def _sc_scatter(base, msg_h, dst_i):
    f = pl.kernel(
        functools.partial(_scatter_body, base),
        out_type=jax.ShapeDtypeStruct((NC, N_NODES, ATOM_FEA), jnp.float32),
        mesh=_mesh(),
        scratch_types=[
            pltpu.VMEM((2, SB), jnp.int32),
            pltpu.VMEM((2, SB, ATOM_FEA), jnp.float32),
            pltpu.VMEM((200, ATOM_FEA), jnp.float32),
            pltpu.VMEM_SHARED((N_NODES, ATOM_FEA), jnp.float32),
            [pltpu.SemaphoreType.DMA, pltpu.SemaphoreType.DMA],
            [pltpu.SemaphoreType.DMA, pltpu.SemaphoreType.DMA],
        ],
    )
    return f(msg_h, dst_i)


def _final_body(u1_ref, u2_ref, a_ref, g2_ref, b2_ref, out_ref):
    u = (u1_ref[0] + u1_ref[1]) + (u2_ref[0] + u2_ref[1])
    n = jnp.float32(N_NODES)
    m = jnp.sum(u, axis=0, keepdims=True) / n
    v = jnp.sum(u * u, axis=0, keepdims=True) / n - m * m
    un = (u - m) * lax.rsqrt(v + EPS) * g2_ref[...] + b2_ref[...]
    z = a_ref[...] + un
    out_ref[...] = jnp.maximum(z, 0.0) + jnp.log(1.0 + jnp.exp(-jnp.abs(z)))


def _final(u1, u2, atom_in_fea, g2, b2):
    return pl.pallas_call(
        _final_body,
        grid=(1,),
        in_specs=[
            pl.BlockSpec((NC, N_NODES, ATOM_FEA), lambda i: (0, 0, 0)),
            pl.BlockSpec((NC, N_NODES, ATOM_FEA), lambda i: (0, 0, 0)),
            pl.BlockSpec((N_NODES, ATOM_FEA), lambda i: (0, 0)),
            pl.BlockSpec((1, ATOM_FEA), lambda i: (0, 0)),
            pl.BlockSpec((1, ATOM_FEA), lambda i: (0, 0)),
        ],
        out_specs=pl.BlockSpec((N_NODES, ATOM_FEA), lambda i: (0, 0)),
        out_shape=jax.ShapeDtypeStruct((N_NODES, ATOM_FEA), jnp.float32),
    )(u1, u2, atom_in_fea, g2, b2)


def kernel(atom_in_fea, nbr_fea, edge_src, edge_dst, W, b,
           gamma1, beta1, gamma2, beta2):
    src_i = edge_src.astype(jnp.int32)
    dst_i = edge_dst.astype(jnp.int32)
    wn = W[:, 2 * ATOM_FEA:]
    b2d = b.reshape(1, OUT_DIM)
    off2 = E_HALF // EB

    ps, pd = _prep(atom_in_fea, W)
    xp1 = _sc_gather(0, ps, pd, src_i, dst_i)
    xp2 = _sc_gather(E_HALF, ps, pd, src_i, dst_i)
    st1 = _stats(xp1, nbr_fea, wn, b2d, 0)
    st2 = _stats(xp2, nbr_fea, wn, b2d, off2)
    sums = st1 + st2

    e = jnp.float32(N_EDGES)
    mean = sums[0] / e
    var = sums[1] / e - mean * mean
    a1 = gamma1 * lax.rsqrt(var + EPS)
    c1 = beta1 + (b - mean) * a1
    a1r = a1.reshape(1, OUT_DIM)
    c1r = c1.reshape(1, OUT_DIM)

    m1 = _normact(xp1, nbr_fea, wn, a1r, c1r, 0)
    u1 = _sc_scatter(0, m1, dst_i)
    m2 = _normact(xp2, nbr_fea, wn, a1r, c1r, off2)
    u2 = _sc_scatter(E_HALF, m2, dst_i)
    return _final(u1, u2, atom_in_fea, gamma2.reshape(1, ATOM_FEA),
                  beta2.reshape(1, ATOM_FEA))


# speedup vs baseline: 1.2095x; 1.0752x over previous
import functools

import jax
import jax.numpy as jnp
from jax import lax
from jax.experimental import pallas as pl
from jax.experimental.pallas import tpu as pltpu
from jax.experimental.pallas import tpu_sc as plsc

ATOM_FEA = 128
NBR_FEA = 16
N_NODES = 10000
N_EDGES = 320000
OUT_DIM = 2 * ATOM_FEA
EPS = 1e-5

NC, NS = 2, 16
NW = NC * NS
E_HALF = N_EDGES // 2
EPW = E_HALF // NW

GB = 40
N_GCHUNK = EPW // GB
SB = 40
N_SCHUNK = EPW // SB

NODE_BLK = 1000
EB = 2000
N_EBLK_H = E_HALF // EB


def _mesh():
    return plsc.VectorSubcoreMesh(core_axis_name="c", subcore_axis_name="s",
                                  num_cores=NC, num_subcores=NS)


def _bf16_bits(x):
    u = lax.bitcast_convert_type(x, jnp.uint32)
    r = u + jnp.uint32(0x7FFF) + ((u >> jnp.uint32(16)) & jnp.uint32(1))
    return r >> jnp.uint32(16)


def _prep_body(a_ref, wse_ref, wso_ref, wde_ref, wdo_ref, ps_ref, pd_ref):
    a = a_ref[...]
    dn = (((1,), (1,)), ((), ()))

    def proj_pack(we, wo):
        e = lax.dot_general(a, we, dn, preferred_element_type=jnp.float32)
        o = lax.dot_general(a, wo, dn, preferred_element_type=jnp.float32)
        packed = _bf16_bits(e) | (_bf16_bits(o) << jnp.uint32(16))
        return lax.bitcast_convert_type(packed, jnp.int32)

    ps_ref[...] = proj_pack(wse_ref[...], wso_ref[...])
    pd_ref[...] = proj_pack(wde_ref[...], wdo_ref[...])


def _prep(atom_in_fea, wse, wso, wde, wdo):
    return pl.pallas_call(
        _prep_body,
        grid=(N_NODES // NODE_BLK,),
        in_specs=[
            pl.BlockSpec((NODE_BLK, ATOM_FEA), lambda i: (i, 0)),
            pl.BlockSpec((ATOM_FEA, ATOM_FEA), lambda i: (0, 0)),
            pl.BlockSpec((ATOM_FEA, ATOM_FEA), lambda i: (0, 0)),
            pl.BlockSpec((ATOM_FEA, ATOM_FEA), lambda i: (0, 0)),
            pl.BlockSpec((ATOM_FEA, ATOM_FEA), lambda i: (0, 0)),
        ],
        out_specs=[
            pl.BlockSpec((NODE_BLK, OUT_DIM // 2), lambda i: (i, 0)),
            pl.BlockSpec((NODE_BLK, OUT_DIM // 2), lambda i: (i, 0)),
        ],
        out_shape=[jax.ShapeDtypeStruct((N_NODES, OUT_DIM // 2),
                                        jnp.int32)] * 2,
    )(atom_in_fea, wse, wso, wde, wdo)


def _gather_body(base, ps_hbm, pd_hbm, src_hbm, dst_hbm, gs_hbm, gd_hbm,
                 si_v, di_v, rs, rd, sem_s, sem_d, sem_ws, sem_wd):
    wid = lax.axis_index("s") * NC + lax.axis_index("c")
    base0 = base + wid * EPW
    obase = wid * EPW
    pltpu.sync_copy(src_hbm.at[pl.ds(base0, EPW)], si_v)
    pltpu.sync_copy(dst_hbm.at[pl.ds(base0, EPW)], di_v)

    def issue(j, p):
        off = j * GB
        pltpu.async_copy(ps_hbm.at[si_v.at[pl.ds(off, GB)]], rs.at[p], sem_s[p])
        pltpu.async_copy(pd_hbm.at[di_v.at[pl.ds(off, GB)]], rd.at[p], sem_d[p])

    def wait_writes(q):
        pltpu.make_async_copy(rs.at[q], gs_hbm.at[pl.ds(0, GB)],
                              sem_ws[q]).wait()
        pltpu.make_async_copy(rd.at[q], gd_hbm.at[pl.ds(0, GB)],
                              sem_wd[q]).wait()

    issue(0, 0)
    issue(1, 1)

    def step(j, p):
        pltpu.make_async_copy(ps_hbm.at[si_v.at[pl.ds(0, GB)]],
                              rs.at[p], sem_s[p]).wait()
        pltpu.make_async_copy(pd_hbm.at[di_v.at[pl.ds(0, GB)]],
                              rd.at[p], sem_d[p]).wait()
        pltpu.async_copy(rs.at[p], gs_hbm.at[pl.ds(obase + j * GB, GB)],
                         sem_ws[p])
        pltpu.async_copy(rd.at[p], gd_hbm.at[pl.ds(obase + j * GB, GB)],
                         sem_wd[p])
        q = (p + 2) % 4

        @pl.when(j >= 2)
        def _():
            wait_writes(q)

        @pl.when(j + 2 < N_GCHUNK)
        def _():
            issue(j + 2, q)

    def quad(k, carry):
        for p in range(4):
            step(4 * k + p, p)
        return carry

    lax.fori_loop(0, N_GCHUNK // 4, quad, 0)
    step(N_GCHUNK - 1, (N_GCHUNK - 1) % 4)
    wait_writes((N_GCHUNK - 1) % 4)
    wait_writes((N_GCHUNK - 2) % 4)


def _sc_gather(base, ps, pd, src_i, dst_i):
    f = pl.kernel(
        functools.partial(_gather_body, base),
        out_type=[jax.ShapeDtypeStruct((E_HALF, OUT_DIM // 2), jnp.int32)] * 2,
        mesh=_mesh(),
        scratch_types=[
            pltpu.VMEM((EPW,), jnp.int32),
            pltpu.VMEM((EPW,), jnp.int32),
            pltpu.VMEM((4, GB, OUT_DIM // 2), jnp.int32),
            pltpu.VMEM((4, GB, OUT_DIM // 2), jnp.int32),
            [pltpu.SemaphoreType.DMA] * 4,
            [pltpu.SemaphoreType.DMA] * 4,
            [pltpu.SemaphoreType.DMA] * 4,
            [pltpu.SemaphoreType.DMA] * 4,
        ],
    )
    return f(ps, pd, src_i, dst_i)


def _unpack_xp(xp_i32):
    u = lax.bitcast_convert_type(xp_i32, jnp.uint32)
    xe = lax.bitcast_convert_type(u << jnp.uint32(16), jnp.float32)
    xo = lax.bitcast_convert_type(u & jnp.uint32(0xFFFF0000), jnp.float32)
    return jnp.concatenate([xe, xo], axis=1)


def _stats_body(xs_ref, xd_ref, nbr_ref, wn_ref, b_ref, s_ref):
    i = pl.program_id(0)
    dn = (((1,), (1,)), ((), ()))
    nproj = lax.dot_general(nbr_ref[...], wn_ref[...], dn,
                            preferred_element_type=jnp.float32)
    x = (_unpack_xp(xs_ref[...]) + _unpack_xp(xd_ref[...])
         + nproj + b_ref[...])
    blk = jnp.concatenate(
        [jnp.sum(x, axis=0, keepdims=True),
         jnp.sum(x * x, axis=0, keepdims=True)], axis=0)

    @pl.when(i == 0)
    def _():
        s_ref[...] = blk

    @pl.when(i > 0)
    def _():
        s_ref[...] = s_ref[...] + blk


def _stats(g_h, nbr, wn, b2d, off_blk):
    return pl.pallas_call(
        _stats_body,
        grid=(N_EBLK_H,),
        in_specs=[
            pl.BlockSpec((EB, OUT_DIM // 2), lambda i: (i, 0)),
            pl.BlockSpec((EB, OUT_DIM // 2), lambda i: (i, 0)),
            pl.BlockSpec((EB, NBR_FEA), lambda i, o=off_blk: (i + o, 0)),
            pl.BlockSpec((OUT_DIM, NBR_FEA), lambda i: (0, 0)),
            pl.BlockSpec((1, OUT_DIM), lambda i: (0, 0)),
        ],
        out_specs=pl.BlockSpec((2, OUT_DIM), lambda i: (0, 0)),
        out_shape=jax.ShapeDtypeStruct((2, OUT_DIM), jnp.float32),
    )(g_h[0], g_h[1], nbr, wn, b2d)


def _normact_body(xs_ref, xd_ref, nbr_ref, wn_ref, a1_ref, c1_ref, msg_ref):
    dn = (((1,), (1,)), ((), ()))
    nproj = lax.dot_general(nbr_ref[...], wn_ref[...], dn,
                            preferred_element_type=jnp.float32)
    y = ((_unpack_xp(xs_ref[...]) + _unpack_xp(xd_ref[...]) + nproj)
         * a1_ref[...] + c1_ref[...])
    f = jnp.concatenate([y[:, :64], y[:, 128:192]], axis=1)
    c = jnp.concatenate([y[:, 64:128], y[:, 192:]], axis=1)
    tf = jnp.exp(-jnp.abs(f))
    sig = jnp.where(f >= 0, 1.0 / (1.0 + tf), tf / (1.0 + tf))
    sp = jnp.maximum(c, 0.0) + jnp.log(1.0 + jnp.exp(-jnp.abs(c)))
    msg_ref[...] = sig * sp


def _normact(g_h, nbr, wn, a1, c1, off_blk):
    return pl.pallas_call(
        _normact_body,
        grid=(N_EBLK_H,),
        in_specs=[
            pl.BlockSpec((EB, OUT_DIM // 2), lambda i: (i, 0)),
            pl.BlockSpec((EB, OUT_DIM // 2), lambda i: (i, 0)),
            pl.BlockSpec((EB, NBR_FEA), lambda i, o=off_blk: (i + o, 0)),
            pl.BlockSpec((OUT_DIM, NBR_FEA), lambda i: (0, 0)),
            pl.BlockSpec((1, OUT_DIM), lambda i: (0, 0)),
            pl.BlockSpec((1, OUT_DIM), lambda i: (0, 0)),
        ],
        out_specs=pl.BlockSpec((EB, ATOM_FEA), lambda i: (i, 0)),
        out_shape=jax.ShapeDtypeStruct((E_HALF, ATOM_FEA), jnp.float32),
    )(g_h[0], g_h[1], nbr, wn, a1, c1)


def _scatter_body(base, msg_hbm, dst_hbm, out_hbm, di2_v, rows_v, zbuf_v,
                  acc_sp, sem_m, sem_i):
    cid = lax.axis_index("c")
    tid = lax.axis_index("s")
    wid = tid * NC + cid
    mbase0 = wid * EPW
    dbase0 = base + wid * EPW

    def zrow(r, c2):
        for c in range(ATOM_FEA // 16):
            zbuf_v[r, pl.ds(c * 16, 16)] = jnp.zeros((16,), jnp.float32)
        return c2
    lax.fori_loop(0, 200, zrow, 0)

    @pl.when(tid < 10)
    def _():
        def zc(j, c2):
            pltpu.sync_copy(zbuf_v, acc_sp.at[pl.ds(tid * 1000 + j * 200, 200)])
            return c2
        lax.fori_loop(0, 5, zc, 0)

    plsc.subcore_barrier()

    def issue(j, p):
        pltpu.async_copy(msg_hbm.at[pl.ds(mbase0 + j * SB, SB)],
                         rows_v.at[p], sem_m[p])
        pltpu.async_copy(dst_hbm.at[pl.ds(dbase0 + j * SB, SB)],
                         di2_v.at[p], sem_i[p])

    issue(0, 0)
    issue(1, 1)

    def half(k, p):
        j = 2 * k + p
        pltpu.make_async_copy(msg_hbm.at[pl.ds(0, SB)], rows_v.at[p],
                              sem_m[p]).wait()
        pltpu.make_async_copy(dst_hbm.at[pl.ds(0, SB)], di2_v.at[p],
                              sem_i[p]).wait()
        pltpu.sync_copy(rows_v.at[p], acc_sp.at[di2_v.at[p]], add=True)

        @pl.when(j + 2 < N_SCHUNK)
        def _():
            issue(j + 2, p)

    def pair(k, carry):
        half(k, 0)
        half(k, 1)
        return carry
    lax.fori_loop(0, N_SCHUNK // 2, pair, 0)
    half(N_SCHUNK // 2, 0)

    plsc.subcore_barrier()

    @pl.when(tid < 10)
    def _():
        def dc(j, c2):
            row = tid * 1000 + j * 200
            pltpu.sync_copy(acc_sp.at[pl.ds(row, 200)], zbuf_v)
            pltpu.sync_copy(zbuf_v, out_hbm.at[cid].at[pl.ds(row, 200)])
            return c2
        lax.fori_loop(0, 5, dc, 0)


def _sc_scatter(base, msg_h, dst_i):
    f = pl.kernel(
        functools.partial(_scatter_body, base),
        out_type=jax.ShapeDtypeStruct((NC, N_NODES, ATOM_FEA), jnp.float32),
        mesh=_mesh(),
        scratch_types=[
            pltpu.VMEM((2, SB), jnp.int32),
            pltpu.VMEM((2, SB, ATOM_FEA), jnp.float32),
            pltpu.VMEM((200, ATOM_FEA), jnp.float32),
            pltpu.VMEM_SHARED((N_NODES, ATOM_FEA), jnp.float32),
            [pltpu.SemaphoreType.DMA, pltpu.SemaphoreType.DMA],
            [pltpu.SemaphoreType.DMA, pltpu.SemaphoreType.DMA],
        ],
    )
    return f(msg_h, dst_i)


def _final_body(u1_ref, u2_ref, a_ref, g2_ref, b2_ref, pm_ref, out_ref):
    u = (u1_ref[0] + u1_ref[1]) + (u2_ref[0] + u2_ref[1])
    n = jnp.float32(N_NODES)
    m = jnp.sum(u, axis=0, keepdims=True) / n
    v = jnp.sum(u * u, axis=0, keepdims=True) / n - m * m
    un = (u - m) * lax.rsqrt(v + EPS) * g2_ref[...] + b2_ref[...]
    ul = jnp.dot(un, pm_ref[...], preferred_element_type=jnp.float32)
    z = a_ref[...] + ul
    out_ref[...] = jnp.maximum(z, 0.0) + jnp.log(1.0 + jnp.exp(-jnp.abs(z)))


def _final(u1, u2, atom_in_fea, g2, b2, pm):
    return pl.pallas_call(
        _final_body,
        grid=(1,),
        in_specs=[
            pl.BlockSpec((NC, N_NODES, ATOM_FEA), lambda i: (0, 0, 0)),
            pl.BlockSpec((NC, N_NODES, ATOM_FEA), lambda i: (0, 0, 0)),
            pl.BlockSpec((N_NODES, ATOM_FEA), lambda i: (0, 0)),
            pl.BlockSpec((1, ATOM_FEA), lambda i: (0, 0)),
            pl.BlockSpec((1, ATOM_FEA), lambda i: (0, 0)),
            pl.BlockSpec((ATOM_FEA, ATOM_FEA), lambda i: (0, 0)),
        ],
        out_specs=pl.BlockSpec((N_NODES, ATOM_FEA), lambda i: (0, 0)),
        out_shape=jax.ShapeDtypeStruct((N_NODES, ATOM_FEA), jnp.float32),
    )(u1, u2, atom_in_fea, g2, b2, pm)


def kernel(atom_in_fea, nbr_fea, edge_src, edge_dst, W, b,
           gamma1, beta1, gamma2, beta2):
    import numpy as np
    src_i = edge_src.astype(jnp.int32)
    dst_i = edge_dst.astype(jnp.int32)

    pi = np.concatenate([np.arange(0, OUT_DIM, 2), np.arange(1, OUT_DIM, 2)])
    pi_m = np.concatenate([np.arange(0, ATOM_FEA, 2), np.arange(1, ATOM_FEA, 2)])
    pm_np = np.zeros((ATOM_FEA, ATOM_FEA), np.float32)
    pm_np[np.arange(ATOM_FEA), pi_m] = 1.0
    pm = jnp.asarray(pm_np)

    ws = W[:, :ATOM_FEA]
    wd = W[:, ATOM_FEA:2 * ATOM_FEA]
    wse, wso = ws[0::2], ws[1::2]
    wde, wdo = wd[0::2], wd[1::2]
    wn = W[:, 2 * ATOM_FEA:][pi]
    b_p = b[pi]
    g1_p = gamma1[pi]
    be1_p = beta1[pi]
    g2_p = gamma2[pi_m].reshape(1, ATOM_FEA)
    be2_p = beta2[pi_m].reshape(1, ATOM_FEA)
    b2d = b_p.reshape(1, OUT_DIM)
    off2 = E_HALF // EB

    ps, pd = _prep(atom_in_fea, wse, wso, wde, wdo)
    xp1 = _sc_gather(0, ps, pd, src_i, dst_i)
    xp2 = _sc_gather(E_HALF, ps, pd, src_i, dst_i)
    st1 = _stats(xp1, nbr_fea, wn, b2d, 0)
    st2 = _stats(xp2, nbr_fea, wn, b2d, off2)
    sums = st1 + st2

    e = jnp.float32(N_EDGES)
    mean = sums[0] / e
    var = sums[1] / e - mean * mean
    a1 = g1_p * lax.rsqrt(var + EPS)
    c1 = be1_p + (b_p - mean) * a1
    a1r = a1.reshape(1, OUT_DIM)
    c1r = c1.reshape(1, OUT_DIM)

    m1 = _normact(xp1, nbr_fea, wn, a1r, c1r, 0)
    u1 = _sc_scatter(0, m1, dst_i)
    m2 = _normact(xp2, nbr_fea, wn, a1r, c1r, off2)
    u2 = _sc_scatter(E_HALF, m2, dst_i)
    return _final(u1, u2, atom_in_fea, g2_p, be2_p, pm)
